# baseline (speedup 1.0000x reference)
import jax
import jax.numpy as jnp
from jax import lax
from jax.experimental import pallas as pl
from jax.experimental.pallas import tpu as pltpu
from jax.experimental.pallas import tpu_sc as plsc

N_NODES = 10000
N_EDGES = 320000
IN_F = 128
HID = 256

NPAD = 10240
NC, NS, L = 2, 16, 16
NW = NC * NS
CHUNK = 128
NCHUNK = 80
EPT = CHUNK * NCHUNK
EPAD = EPT * NW
FC = 128


NBUF = 2

SLOWC = 0
NCH_S = 42
NCH_F = 118
TOTCH = NS * (NCH_S + NCH_F)


def _spmm_body(feat_hbm, src_hbm, dst_hbm, out_hbm, cidx, rows, acc, *sems):
    cid = lax.axis_index("c")
    sid = lax.axis_index("s")
    gsem = sems[:NBUF]
    ssem = sems[NBUF:]
    nch = jnp.where(cid == SLOWC, NCH_S, NCH_F)
    cbase = jnp.where(cid == SLOWC, sid * NCH_S, NS * NCH_S + sid * NCH_F)

    def zfill(i, c):
        def zc(k, c2):
            rows[0, i, pl.ds(k * L, L)] = jnp.zeros((L,), jnp.float32)
            return c2
        lax.fori_loop(0, FC // L, zc, 0)
        return c
    lax.fori_loop(0, CHUNK, zfill, 0)

    rpt = NPAD // NS
    base = sid * rpt

    def zloop(k, c):
        pltpu.sync_copy(rows.at[0], acc.at[pl.ds(base + k * CHUNK, CHUNK)])
        return c
    lax.fori_loop(0, rpt // CHUNK, zloop, 0)
    plsc.subcore_barrier()

    for b in range(NBUF):
        pltpu.sync_copy(src_hbm.at[cbase + b], cidx.at[b, 0])
        pltpu.sync_copy(dst_hbm.at[cbase + b], cidx.at[b, 1])
        pltpu.async_copy(feat_hbm.at[cidx.at[b, 0]], rows.at[b], gsem[b])

    def mloop(t, c):
        for b in range(NBUF):
            pltpu.make_async_copy(
                feat_hbm.at[cidx.at[b, 0]], rows.at[b], gsem[b]).wait()
            pltpu.make_async_copy(
                rows.at[b], acc.at[cidx.at[b, 1]], ssem[b]).start(add=True)
        for b in range(NBUF):
            j = t * NBUF + b
            pltpu.make_async_copy(
                rows.at[b], acc.at[cidx.at[b, 1]], ssem[b]).wait()

            @pl.when(j + NBUF < nch)
            def _():
                pltpu.sync_copy(src_hbm.at[cbase + j + NBUF], cidx.at[b, 0])
                pltpu.sync_copy(dst_hbm.at[cbase + j + NBUF], cidx.at[b, 1])
                pltpu.async_copy(feat_hbm.at[cidx.at[b, 0]], rows.at[b], gsem[b])
        return c
    lax.fori_loop(0, nch // NBUF, mloop, 0)
    plsc.subcore_barrier()

    pltpu.sync_copy(acc.at[pl.ds(base, rpt)], out_hbm.at[cid, pl.ds(base, rpt)])


def _make_spmm():
    return pl.kernel(
        _spmm_body,
        out_type=jax.ShapeDtypeStruct((NC, NPAD, FC), jnp.float32),
        mesh=plsc.VectorSubcoreMesh(core_axis_name="c", subcore_axis_name="s"),
        scratch_types=[
            pltpu.VMEM((NBUF, 2, CHUNK), jnp.int32),
            pltpu.VMEM((NBUF, CHUNK, FC), jnp.float32),
            pltpu.VMEM_SHARED((NPAD, FC), jnp.float32),
        ] + [pltpu.SemaphoreType.DMA] * (2 * NBUF),
    )


RB = 512
NB = NPAD // RB
BE = 512
NEB = N_EDGES // BE


def _deg_body(s_ref, d_ref, os_ref, od_ref, as_sc, ad_sc):
    i = pl.program_id(0)

    @pl.when(i == 0)
    def _():
        as_sc[...] = jnp.zeros((128, 128), jnp.float32)
        ad_sc[...] = jnp.zeros((128, 128), jnp.float32)

    lane = lax.broadcasted_iota(jnp.int32, (1, 128), 1)

    def onehot_acc(ids, acc):
        hi = lax.shift_right_logical(ids, 7)
        lo = lax.bitwise_and(ids, 127)
        oh_hi = (hi == lane).astype(jnp.float32)
        oh_lo = (lo == lane).astype(jnp.float32)
        acc[...] += lax.dot_general(
            oh_hi, oh_lo, (((0,), (0,)), ((), ())),
            preferred_element_type=jnp.float32)

    onehot_acc(s_ref[...], as_sc)
    onehot_acc(d_ref[...], ad_sc)

    @pl.when(i == NEB - 1)
    def _():
        os_ref[...] = as_sc[...]
        od_ref[...] = ad_sc[...]


def _make_deg():
    return pl.pallas_call(
        _deg_body,
        grid=(NEB,),
        in_specs=[
            pl.BlockSpec((BE, 1), lambda i: (i, 0)),
            pl.BlockSpec((BE, 1), lambda i: (i, 0)),
        ],
        out_specs=[
            pl.BlockSpec((128, 128), lambda i: (0, 0)),
            pl.BlockSpec((128, 128), lambda i: (0, 0)),
        ],
        out_shape=[
            jax.ShapeDtypeStruct((128, 128), jnp.float32),
            jax.ShapeDtypeStruct((128, 128), jnp.float32),
        ],
        scratch_shapes=[
            pltpu.VMEM((128, 128), jnp.float32),
            pltpu.VMEM((128, 128), jnp.float32),
        ],
    )


def _norm(d_ref):
    return lax.rsqrt(jnp.maximum(d_ref[...], 1.0))


def _prep_body(x_ref, dsrc_ref, xn_ref):
    xn_ref[...] = x_ref[...] * _norm(dsrc_ref)


def _layer1_body(agg_ref, ddst_ref, dsrc_ref, w_ref, b_ref, ha_ref, hb_ref):
    agg = (agg_ref[0] + agg_ref[1]) * _norm(ddst_ref)
    h = jnp.maximum(jnp.dot(agg, w_ref[...],
                            preferred_element_type=jnp.float32) + b_ref[...], 0.0)
    hn = h * _norm(dsrc_ref)
    ha_ref[...] = hn[:, :FC]
    hb_ref[...] = hn[:, FC:]


def _layer2_pool_body(agga_ref, aggb_ref, ddst_ref, w_ref, b_ref, wg_ref,
                      bg_ref, wo_ref, bo_ref, out_ref, m_sc, s_sc, r_sc):
    i = pl.program_id(0)
    agg = jnp.concatenate([agga_ref[0] + agga_ref[1],
                           aggb_ref[0] + aggb_ref[1]], axis=1) * _norm(ddst_ref)
    h = jnp.maximum(jnp.dot(agg, w_ref[...],
                            preferred_element_type=jnp.float32) + b_ref[...], 0.0)
    g = jnp.sum(h * wg_ref[...], axis=1) + bg_ref[0, 0]
    rid = i * RB + lax.broadcasted_iota(jnp.int32, (RB,), 0)
    g = jnp.where(rid < N_NODES, g, -1e30)

    @pl.when(i == 0)
    def _():
        m_sc[0, 0] = -1e30
        s_sc[0, 0] = 0.0
        r_sc[...] = jnp.zeros((1, HID), jnp.float32)

    m_old = m_sc[0, 0]
    m_new = jnp.maximum(m_old, jnp.max(g))
    alpha = jnp.exp(m_old - m_new)
    e = jnp.exp(g - m_new)
    s_sc[0, 0] = s_sc[0, 0] * alpha + jnp.sum(e)
    r_sc[...] = r_sc[...] * alpha + jnp.dot(
        e[None, :], h, preferred_element_type=jnp.float32)
    m_sc[0, 0] = m_new

    @pl.when(i == NB - 1)
    def _():
        val = jnp.sum(r_sc[...] * wo_ref[...]) / s_sc[0, 0] + bo_ref[0, 0]
        out_ref[...] = jnp.broadcast_to(val, (1, 1))


def _full(shape):
    return pl.BlockSpec(shape, lambda i: tuple(0 for _ in shape))


def _make_prep():
    return pl.pallas_call(
        _prep_body,
        grid=(NB,),
        in_specs=[
            pl.BlockSpec((RB, IN_F), lambda i: (i, 0)),
            pl.BlockSpec((RB, 1), lambda i: (i, 0)),
        ],
        out_specs=pl.BlockSpec((RB, IN_F), lambda i: (i, 0)),
        out_shape=jax.ShapeDtypeStruct((NPAD, IN_F), jnp.float32),
    )


def _make_layer1():
    return pl.pallas_call(
        _layer1_body,
        grid=(NB,),
        in_specs=[
            pl.BlockSpec((NC, RB, FC), lambda i: (0, i, 0)),
            pl.BlockSpec((RB, 1), lambda i: (i, 0)),
            pl.BlockSpec((RB, 1), lambda i: (i, 0)),
            _full((IN_F, HID)),
            _full((1, HID)),
        ],
        out_specs=[
            pl.BlockSpec((RB, FC), lambda i: (i, 0)),
            pl.BlockSpec((RB, FC), lambda i: (i, 0)),
        ],
        out_shape=[
            jax.ShapeDtypeStruct((NPAD, FC), jnp.float32),
            jax.ShapeDtypeStruct((NPAD, FC), jnp.float32),
        ],
    )


def _make_layer2_pool():
    return pl.pallas_call(
        _layer2_pool_body,
        grid=(NB,),
        in_specs=[
            pl.BlockSpec((NC, RB, FC), lambda i: (0, i, 0)),
            pl.BlockSpec((NC, RB, FC), lambda i: (0, i, 0)),
            pl.BlockSpec((RB, 1), lambda i: (i, 0)),
            _full((HID, HID)),
            _full((1, HID)),
            _full((1, HID)),
            _full((1, 1)),
            _full((1, HID)),
            _full((1, 1)),
        ],
        out_specs=pl.BlockSpec((1, 1), lambda i: (0, 0)),
        out_shape=jax.ShapeDtypeStruct((1, 1), jnp.float32),
        scratch_shapes=[
            pltpu.SMEM((1, 1), jnp.float32),
            pltpu.SMEM((1, 1), jnp.float32),
            pltpu.VMEM((1, HID), jnp.float32),
        ],
    )


@jax.jit
def _run(x, src, dst, W1, b1, W2, b2, Wg, bg, Wo, bo):
    pad_e = EPAD - N_EDGES
    fill = jnp.full((pad_e,), N_NODES, jnp.int32)
    src_p = jnp.concatenate([src, fill]).reshape(TOTCH, CHUNK)
    dst_p = jnp.concatenate([dst, fill]).reshape(TOTCH, CHUNK)
    x_p = jnp.pad(x, ((0, NPAD - N_NODES), (0, 0)))

    ds_hl, dd_hl = _make_deg()(src.reshape(N_EDGES, 1), dst.reshape(N_EDGES, 1))
    dsrc = ds_hl.reshape(128 * 128)[:NPAD, None]
    ddst = dd_hl.reshape(128 * 128)[:NPAD, None]

    spmm = _make_spmm()
    xn = _make_prep()(x_p, dsrc)
    agg1 = spmm(xn, src_p, dst_p)
    h1a, h1b = _make_layer1()(agg1, ddst, dsrc, W1, b1.reshape(1, HID))
    agg2a = spmm(h1a, src_p, dst_p)
    h1b_seq, agg2a = lax.optimization_barrier((h1b, agg2a))
    agg2b = spmm(h1b_seq, src_p, dst_p)
    out = _make_layer2_pool()(
        agg2a, agg2b, ddst, W2, b2.reshape(1, HID),
        Wg.reshape(1, HID), bg.reshape(1, 1),
        Wo.reshape(1, HID), bo.reshape(1, 1))
    return out


def kernel(x, edge_index, W1, b1, W2, b2, Wg, bg, Wo, bo):
    ei = edge_index.astype(jnp.int32)
    return _run(x, ei[0], ei[1], W1, b1, W2, b2, Wg, bg, Wo, bo)


# speedup vs baseline: 1.2086x; 1.2086x over previous
import jax
import jax.numpy as jnp
from jax import lax
from jax.experimental import pallas as pl
from jax.experimental.pallas import tpu as pltpu
from jax.experimental.pallas import tpu_sc as plsc

N_NODES = 10000
N_EDGES = 320000
IN_F = 128
HID = 256

NPAD = 10240
NC, NS, L = 2, 16, 16
NW = NC * NS
CHUNK = 128
NCHUNK = 80
EPT = CHUNK * NCHUNK
EPAD = EPT * NW
FC = 128


NBUF = 2

SLOWC = 1
NCH_S = 80
NCH_F = 80
TOTCH = NS * (NCH_S + NCH_F)


def _spmm_body(feat_hbm, idx_hbm, out_hbm, cidx, rows, acc, *sems):
    cid = lax.axis_index("c")
    sid = lax.axis_index("s")
    gsem = sems[:NBUF]
    ssem = sems[NBUF:]
    nch = jnp.where(cid == SLOWC, NCH_S, NCH_F)
    cbase = jnp.where(cid == SLOWC, sid * NCH_S, NS * NCH_S + sid * NCH_F)

    def zfill(i, c):
        def zc(k, c2):
            rows[0, i, pl.ds(k * L, L)] = jnp.zeros((L,), jnp.float32)
            return c2
        lax.fori_loop(0, FC // L, zc, 0)
        return c
    lax.fori_loop(0, CHUNK, zfill, 0)

    rpt = NPAD // NS
    base = sid * rpt

    def zloop(k, c):
        pltpu.sync_copy(rows.at[0], acc.at[pl.ds(base + k * CHUNK, CHUNK)])
        return c
    lax.fori_loop(0, rpt // CHUNK, zloop, 0)
    plsc.subcore_barrier()

    for b in range(NBUF):
        pltpu.sync_copy(idx_hbm.at[cbase + b], cidx.at[b])
        pltpu.async_copy(feat_hbm.at[cidx.at[b, 0]], rows.at[b], gsem[b])

    def mloop(t, c):
        for b in range(NBUF):
            pltpu.make_async_copy(
                feat_hbm.at[cidx.at[b, 0]], rows.at[b], gsem[b]).wait()
            pltpu.make_async_copy(
                rows.at[b], acc.at[cidx.at[b, 1]], ssem[b]).start(add=True)
        for b in range(NBUF):
            j = t * NBUF + b
            pltpu.make_async_copy(
                rows.at[b], acc.at[cidx.at[b, 1]], ssem[b]).wait()

            @pl.when(j + NBUF < nch)
            def _():
                pltpu.sync_copy(idx_hbm.at[cbase + j + NBUF], cidx.at[b])
                pltpu.async_copy(feat_hbm.at[cidx.at[b, 0]], rows.at[b], gsem[b])
        return c
    lax.fori_loop(0, nch // NBUF, mloop, 0)
    plsc.subcore_barrier()

    pltpu.sync_copy(acc.at[pl.ds(base, rpt)], out_hbm.at[cid, pl.ds(base, rpt)])


def _make_spmm():
    return pl.kernel(
        _spmm_body,
        out_type=jax.ShapeDtypeStruct((NC, NPAD, FC), jnp.float32),
        mesh=plsc.VectorSubcoreMesh(core_axis_name="c", subcore_axis_name="s"),
        scratch_types=[
            pltpu.VMEM((NBUF, 2, CHUNK), jnp.int32),
            pltpu.VMEM((NBUF, CHUNK, FC), jnp.float32),
            pltpu.VMEM_SHARED((NPAD, FC), jnp.float32),
        ] + [pltpu.SemaphoreType.DMA] * (2 * NBUF),
    )


RB = 512
NB = NPAD // RB
BE = 512
NEB = N_EDGES // BE


def _deg_body(s_ref, d_ref, os_ref, od_ref, as_sc, ad_sc):
    i = pl.program_id(0)

    @pl.when(i == 0)
    def _():
        as_sc[...] = jnp.zeros((128, 128), jnp.float32)
        ad_sc[...] = jnp.zeros((128, 128), jnp.float32)

    lane = lax.broadcasted_iota(jnp.int32, (1, 128), 1)

    def onehot_acc(ids, acc):
        hi = lax.shift_right_logical(ids, 7)
        lo = lax.bitwise_and(ids, 127)
        oh_hi = (hi == lane).astype(jnp.float32)
        oh_lo = (lo == lane).astype(jnp.float32)
        acc[...] += lax.dot_general(
            oh_hi, oh_lo, (((0,), (0,)), ((), ())),
            preferred_element_type=jnp.float32)

    onehot_acc(s_ref[...], as_sc)
    onehot_acc(d_ref[...], ad_sc)

    @pl.when(i == NEB - 1)
    def _():
        os_ref[...] = as_sc[...]
        od_ref[...] = ad_sc[...]


def _make_deg():
    return pl.pallas_call(
        _deg_body,
        grid=(NEB,),
        in_specs=[
            pl.BlockSpec((BE, 1), lambda i: (i, 0)),
            pl.BlockSpec((BE, 1), lambda i: (i, 0)),
        ],
        out_specs=[
            pl.BlockSpec((128, 128), lambda i: (0, 0)),
            pl.BlockSpec((128, 128), lambda i: (0, 0)),
        ],
        out_shape=[
            jax.ShapeDtypeStruct((128, 128), jnp.float32),
            jax.ShapeDtypeStruct((128, 128), jnp.float32),
        ],
        scratch_shapes=[
            pltpu.VMEM((128, 128), jnp.float32),
            pltpu.VMEM((128, 128), jnp.float32),
        ],
    )


def _norm(d_ref):
    return lax.rsqrt(jnp.maximum(d_ref[...], 1.0))


def _prep_body(x_ref, dsrc_ref, xn_ref):
    xn_ref[...] = x_ref[...] * _norm(dsrc_ref)


def _layer1_body(agg_ref, ddst_ref, dsrc_ref, w_ref, b_ref, ha_ref, hb_ref):
    agg = (agg_ref[0] + agg_ref[1]) * _norm(ddst_ref)
    h = jnp.maximum(jnp.dot(agg, w_ref[...],
                            preferred_element_type=jnp.float32) + b_ref[...], 0.0)
    hn = h * _norm(dsrc_ref)
    ha_ref[...] = hn[:, :FC]
    hb_ref[...] = hn[:, FC:]


def _layer2_pool_body(agga_ref, aggb_ref, ddst_ref, w_ref, b_ref, wg_ref,
                      bg_ref, wo_ref, bo_ref, out_ref, m_sc, s_sc, r_sc):
    i = pl.program_id(0)
    agg = jnp.concatenate([agga_ref[0] + agga_ref[1],
                           aggb_ref[0] + aggb_ref[1]], axis=1) * _norm(ddst_ref)
    h = jnp.maximum(jnp.dot(agg, w_ref[...],
                            preferred_element_type=jnp.float32) + b_ref[...], 0.0)
    g = jnp.sum(h * wg_ref[...], axis=1) + bg_ref[0, 0]
    rid = i * RB + lax.broadcasted_iota(jnp.int32, (RB,), 0)
    g = jnp.where(rid < N_NODES, g, -1e30)

    @pl.when(i == 0)
    def _():
        m_sc[0, 0] = -1e30
        s_sc[0, 0] = 0.0
        r_sc[...] = jnp.zeros((1, HID), jnp.float32)

    m_old = m_sc[0, 0]
    m_new = jnp.maximum(m_old, jnp.max(g))
    alpha = jnp.exp(m_old - m_new)
    e = jnp.exp(g - m_new)
    s_sc[0, 0] = s_sc[0, 0] * alpha + jnp.sum(e)
    r_sc[...] = r_sc[...] * alpha + jnp.dot(
        e[None, :], h, preferred_element_type=jnp.float32)
    m_sc[0, 0] = m_new

    @pl.when(i == NB - 1)
    def _():
        val = jnp.sum(r_sc[...] * wo_ref[...]) / s_sc[0, 0] + bo_ref[0, 0]
        out_ref[...] = jnp.broadcast_to(val, (1, 1))


def _full(shape):
    return pl.BlockSpec(shape, lambda i: tuple(0 for _ in shape))


def _make_prep():
    return pl.pallas_call(
        _prep_body,
        grid=(NB,),
        in_specs=[
            pl.BlockSpec((RB, IN_F), lambda i: (i, 0)),
            pl.BlockSpec((RB, 1), lambda i: (i, 0)),
        ],
        out_specs=pl.BlockSpec((RB, IN_F), lambda i: (i, 0)),
        out_shape=jax.ShapeDtypeStruct((NPAD, IN_F), jnp.float32),
    )


def _make_layer1():
    return pl.pallas_call(
        _layer1_body,
        grid=(NB,),
        in_specs=[
            pl.BlockSpec((NC, RB, FC), lambda i: (0, i, 0)),
            pl.BlockSpec((RB, 1), lambda i: (i, 0)),
            pl.BlockSpec((RB, 1), lambda i: (i, 0)),
            _full((IN_F, HID)),
            _full((1, HID)),
        ],
        out_specs=[
            pl.BlockSpec((RB, FC), lambda i: (i, 0)),
            pl.BlockSpec((RB, FC), lambda i: (i, 0)),
        ],
        out_shape=[
            jax.ShapeDtypeStruct((NPAD, FC), jnp.float32),
            jax.ShapeDtypeStruct((NPAD, FC), jnp.float32),
        ],
    )


def _make_layer2_pool():
    return pl.pallas_call(
        _layer2_pool_body,
        grid=(NB,),
        in_specs=[
            pl.BlockSpec((NC, RB, FC), lambda i: (0, i, 0)),
            pl.BlockSpec((NC, RB, FC), lambda i: (0, i, 0)),
            pl.BlockSpec((RB, 1), lambda i: (i, 0)),
            _full((HID, HID)),
            _full((1, HID)),
            _full((1, HID)),
            _full((1, 1)),
            _full((1, HID)),
            _full((1, 1)),
        ],
        out_specs=pl.BlockSpec((1, 1), lambda i: (0, 0)),
        out_shape=jax.ShapeDtypeStruct((1, 1), jnp.float32),
        scratch_shapes=[
            pltpu.SMEM((1, 1), jnp.float32),
            pltpu.SMEM((1, 1), jnp.float32),
            pltpu.VMEM((1, HID), jnp.float32),
        ],
    )


@jax.jit
def _run(x, src, dst, W1, b1, W2, b2, Wg, bg, Wo, bo):
    pad_e = EPAD - N_EDGES
    fill = jnp.full((pad_e,), N_NODES, jnp.int32)
    src_p = jnp.concatenate([src, fill]).reshape(TOTCH, 1, CHUNK)
    dst_p = jnp.concatenate([dst, fill]).reshape(TOTCH, 1, CHUNK)
    idx_p = jnp.concatenate([src_p, dst_p], axis=1)
    x_p = jnp.pad(x, ((0, NPAD - N_NODES), (0, 0)))

    ds_hl, dd_hl = _make_deg()(src.reshape(N_EDGES, 1), dst.reshape(N_EDGES, 1))
    dsrc = ds_hl.reshape(128 * 128)[:NPAD, None]
    ddst = dd_hl.reshape(128 * 128)[:NPAD, None]

    spmm = _make_spmm()
    xn = _make_prep()(x_p, dsrc)
    agg1 = spmm(xn, idx_p)
    h1a, h1b = _make_layer1()(agg1, ddst, dsrc, W1, b1.reshape(1, HID))
    agg2a = spmm(h1a, idx_p)
    h1b_seq, agg2a = lax.optimization_barrier((h1b, agg2a))
    agg2b = spmm(h1b_seq, idx_p)
    out = _make_layer2_pool()(
        agg2a, agg2b, ddst, W2, b2.reshape(1, HID),
        Wg.reshape(1, HID), bg.reshape(1, 1),
        Wo.reshape(1, HID), bo.reshape(1, 1))
    return out


def kernel(x, edge_index, W1, b1, W2, b2, Wg, bg, Wo, bo):
    ei = edge_index.astype(jnp.int32)
    return _run(x, ei[0], ei[1], W1, b1, W2, b2, Wg, bg, Wo, bo)


# speedup vs baseline: 1.3907x; 1.1507x over previous
import jax
import jax.numpy as jnp
from jax import lax
from jax.experimental import pallas as pl
from jax.experimental.pallas import tpu as pltpu
from jax.experimental.pallas import tpu_sc as plsc

N_NODES = 10000
N_EDGES = 320000
IN_F = 128
HID = 256

NPAD = 10240
NC, NS, L = 2, 16, 16
NW = NC * NS
CHUNK = 128
NCHUNK = 80
EPT = CHUNK * NCHUNK
EPAD = EPT * NW
FC = 128


NBUF = 2

SLOWC = 1
NCH_S = 80
NCH_F = 80
TOTCH = NS * (NCH_S + NCH_F)


def _spmm_body(feat_hbm, idx_hbm, out_hbm, cidx, rows, acc, *sems):
    cid = lax.axis_index("c")
    sid = lax.axis_index("s")
    gsem = sems[:NBUF]
    ssem = sems[NBUF:]
    nch = jnp.where(cid == SLOWC, NCH_S, NCH_F)
    cbase = jnp.where(cid == SLOWC, sid * NCH_S, NS * NCH_S + sid * NCH_F)

    def zfill(i, c):
        def zc(k, c2):
            rows[0, i, pl.ds(k * L, L)] = jnp.zeros((L,), jnp.float32)
            return c2
        lax.fori_loop(0, FC // L, zc, 0)
        return c
    lax.fori_loop(0, CHUNK, zfill, 0)

    rpt = NPAD // NS
    base = sid * rpt

    zd = [pltpu.make_async_copy(
        rows.at[0], acc.at[pl.ds(base + k * CHUNK, CHUNK)], ssem[0])
        for k in range(rpt // CHUNK)]
    for d in zd:
        d.start()
    for d in zd:
        d.wait()
    plsc.subcore_barrier()

    for b in range(NBUF):
        pltpu.sync_copy(idx_hbm.at[cbase + b], cidx.at[b])
        pltpu.async_copy(feat_hbm.at[cidx.at[b, 0]], rows.at[b], gsem[b])

    def mloop(t, c):
        for b in range(NBUF):
            pltpu.make_async_copy(
                feat_hbm.at[cidx.at[b, 0]], rows.at[b], gsem[b]).wait()
            pltpu.make_async_copy(
                rows.at[b], acc.at[cidx.at[b, 1]], ssem[b]).start(add=True)
        for b in range(NBUF):
            j = t * NBUF + b
            pltpu.make_async_copy(
                rows.at[b], acc.at[cidx.at[b, 1]], ssem[b]).wait()

            @pl.when(j + NBUF < nch)
            def _():
                pltpu.sync_copy(idx_hbm.at[cbase + j + NBUF], cidx.at[b])
                pltpu.async_copy(feat_hbm.at[cidx.at[b, 0]], rows.at[b], gsem[b])
        return c
    lax.fori_loop(0, nch // NBUF, mloop, 0)
    plsc.subcore_barrier()

    pltpu.sync_copy(acc.at[pl.ds(base, rpt)], out_hbm.at[cid, pl.ds(base, rpt)])


def _make_spmm():
    return pl.kernel(
        _spmm_body,
        out_type=jax.ShapeDtypeStruct((NC, NPAD, FC), jnp.float32),
        mesh=plsc.VectorSubcoreMesh(core_axis_name="c", subcore_axis_name="s"),
        scratch_types=[
            pltpu.VMEM((NBUF, 2, CHUNK), jnp.int32),
            pltpu.VMEM((NBUF, CHUNK, FC), jnp.float32),
            pltpu.VMEM_SHARED((NPAD, FC), jnp.float32),
        ] + [pltpu.SemaphoreType.DMA] * (2 * NBUF),
    )


RB = 512
NB = NPAD // RB
BE = 2560
NEB = N_EDGES // BE


def _deg_body(s_ref, d_ref, os_ref, od_ref, as_sc, ad_sc):
    i = pl.program_id(0)

    @pl.when(i == 0)
    def _():
        as_sc[...] = jnp.zeros((128, 128), jnp.float32)
        ad_sc[...] = jnp.zeros((128, 128), jnp.float32)

    lane = lax.broadcasted_iota(jnp.int32, (1, 128), 1)

    def onehot_acc(ids, acc):
        hi = lax.shift_right_logical(ids, 7)
        lo = lax.bitwise_and(ids, 127)
        oh_hi = (hi == lane).astype(jnp.float32)
        oh_lo = (lo == lane).astype(jnp.float32)
        acc[...] += lax.dot_general(
            oh_hi, oh_lo, (((0,), (0,)), ((), ())),
            preferred_element_type=jnp.float32)

    onehot_acc(s_ref[...], as_sc)
    onehot_acc(d_ref[...], ad_sc)

    @pl.when(i == NEB - 1)
    def _():
        os_ref[...] = as_sc[...]
        od_ref[...] = ad_sc[...]


def _make_deg():
    return pl.pallas_call(
        _deg_body,
        grid=(NEB,),
        in_specs=[
            pl.BlockSpec((BE, 1), lambda i: (i, 0)),
            pl.BlockSpec((BE, 1), lambda i: (i, 0)),
        ],
        out_specs=[
            pl.BlockSpec((128, 128), lambda i: (0, 0)),
            pl.BlockSpec((128, 128), lambda i: (0, 0)),
        ],
        out_shape=[
            jax.ShapeDtypeStruct((128, 128), jnp.float32),
            jax.ShapeDtypeStruct((128, 128), jnp.float32),
        ],
        scratch_shapes=[
            pltpu.VMEM((128, 128), jnp.float32),
            pltpu.VMEM((128, 128), jnp.float32),
        ],
    )


def _norm(d_ref):
    return lax.rsqrt(jnp.maximum(d_ref[...], 1.0))


def _prep_body(x_ref, dsrc_ref, xn_ref):
    xn_ref[...] = x_ref[...] * _norm(dsrc_ref)


def _layer1_body(agg_ref, ddst_ref, dsrc_ref, w_ref, b_ref, ha_ref, hb_ref):
    agg = (agg_ref[0] + agg_ref[1]) * _norm(ddst_ref)
    h = jnp.maximum(jnp.dot(agg, w_ref[...],
                            preferred_element_type=jnp.float32) + b_ref[...], 0.0)
    hn = h * _norm(dsrc_ref)
    ha_ref[...] = hn[:, :FC]
    hb_ref[...] = hn[:, FC:]


def _layer2_pool_body(agga_ref, aggb_ref, ddst_ref, w_ref, b_ref, wg_ref,
                      bg_ref, wo_ref, bo_ref, out_ref, m_sc, s_sc, r_sc):
    i = pl.program_id(0)
    agg = jnp.concatenate([agga_ref[0] + agga_ref[1],
                           aggb_ref[0] + aggb_ref[1]], axis=1) * _norm(ddst_ref)
    h = jnp.maximum(jnp.dot(agg, w_ref[...],
                            preferred_element_type=jnp.float32) + b_ref[...], 0.0)
    g = jnp.sum(h * wg_ref[...], axis=1) + bg_ref[0, 0]
    rid = i * RB + lax.broadcasted_iota(jnp.int32, (RB,), 0)
    g = jnp.where(rid < N_NODES, g, -1e30)

    @pl.when(i == 0)
    def _():
        m_sc[0, 0] = -1e30
        s_sc[0, 0] = 0.0
        r_sc[...] = jnp.zeros((1, HID), jnp.float32)

    m_old = m_sc[0, 0]
    m_new = jnp.maximum(m_old, jnp.max(g))
    alpha = jnp.exp(m_old - m_new)
    e = jnp.exp(g - m_new)
    s_sc[0, 0] = s_sc[0, 0] * alpha + jnp.sum(e)
    r_sc[...] = r_sc[...] * alpha + jnp.dot(
        e[None, :], h, preferred_element_type=jnp.float32)
    m_sc[0, 0] = m_new

    @pl.when(i == NB - 1)
    def _():
        val = jnp.sum(r_sc[...] * wo_ref[...]) / s_sc[0, 0] + bo_ref[0, 0]
        out_ref[...] = jnp.broadcast_to(val, (1, 1))


def _full(shape):
    return pl.BlockSpec(shape, lambda i: tuple(0 for _ in shape))


def _make_prep():
    return pl.pallas_call(
        _prep_body,
        grid=(NB,),
        in_specs=[
            pl.BlockSpec((RB, IN_F), lambda i: (i, 0)),
            pl.BlockSpec((RB, 1), lambda i: (i, 0)),
        ],
        out_specs=pl.BlockSpec((RB, IN_F), lambda i: (i, 0)),
        out_shape=jax.ShapeDtypeStruct((NPAD, IN_F), jnp.float32),
    )


def _make_layer1():
    return pl.pallas_call(
        _layer1_body,
        grid=(NB,),
        in_specs=[
            pl.BlockSpec((NC, RB, FC), lambda i: (0, i, 0)),
            pl.BlockSpec((RB, 1), lambda i: (i, 0)),
            pl.BlockSpec((RB, 1), lambda i: (i, 0)),
            _full((IN_F, HID)),
            _full((1, HID)),
        ],
        out_specs=[
            pl.BlockSpec((RB, FC), lambda i: (i, 0)),
            pl.BlockSpec((RB, FC), lambda i: (i, 0)),
        ],
        out_shape=[
            jax.ShapeDtypeStruct((NPAD, FC), jnp.float32),
            jax.ShapeDtypeStruct((NPAD, FC), jnp.float32),
        ],
    )


def _make_layer2_pool():
    return pl.pallas_call(
        _layer2_pool_body,
        grid=(NB,),
        in_specs=[
            pl.BlockSpec((NC, RB, FC), lambda i: (0, i, 0)),
            pl.BlockSpec((NC, RB, FC), lambda i: (0, i, 0)),
            pl.BlockSpec((RB, 1), lambda i: (i, 0)),
            _full((HID, HID)),
            _full((1, HID)),
            _full((1, HID)),
            _full((1, 1)),
            _full((1, HID)),
            _full((1, 1)),
        ],
        out_specs=pl.BlockSpec((1, 1), lambda i: (0, 0)),
        out_shape=jax.ShapeDtypeStruct((1, 1), jnp.float32),
        scratch_shapes=[
            pltpu.SMEM((1, 1), jnp.float32),
            pltpu.SMEM((1, 1), jnp.float32),
            pltpu.VMEM((1, HID), jnp.float32),
        ],
    )


@jax.jit
def _run(x, src, dst, W1, b1, W2, b2, Wg, bg, Wo, bo):
    pad_e = EPAD - N_EDGES
    fill = jnp.full((pad_e,), N_NODES, jnp.int32)
    src_p = jnp.concatenate([src, fill]).reshape(TOTCH, 1, CHUNK)
    dst_p = jnp.concatenate([dst, fill]).reshape(TOTCH, 1, CHUNK)
    idx_p = jnp.concatenate([src_p, dst_p], axis=1)
    x_p = jnp.pad(x, ((0, NPAD - N_NODES), (0, 0)))

    ds_hl, dd_hl = _make_deg()(src.reshape(N_EDGES, 1), dst.reshape(N_EDGES, 1))
    dsrc = ds_hl.reshape(128 * 128)[:NPAD, None]
    ddst = dd_hl.reshape(128 * 128)[:NPAD, None]

    spmm = _make_spmm()
    xn = _make_prep()(x_p, dsrc)
    agg1 = spmm(xn, idx_p)
    h1a, h1b = _make_layer1()(agg1, ddst, dsrc, W1, b1.reshape(1, HID))
    agg2a = spmm(h1a, idx_p)
    h1b_seq, agg2a = lax.optimization_barrier((h1b, agg2a))
    agg2b = spmm(h1b_seq, idx_p)
    out = _make_layer2_pool()(
        agg2a, agg2b, ddst, W2, b2.reshape(1, HID),
        Wg.reshape(1, HID), bg.reshape(1, 1),
        Wo.reshape(1, HID), bo.reshape(1, 1))
    return out


def kernel(x, edge_index, W1, b1, W2, b2, Wg, bg, Wo, bo):
    ei = edge_index.astype(jnp.int32)
    return _run(x, ei[0], ei[1], W1, b1, W2, b2, Wg, bg, Wo, bo)


# speedup vs baseline: 1.4092x; 1.0133x over previous
import jax
import jax.numpy as jnp
from jax import lax
from jax.experimental import pallas as pl
from jax.experimental.pallas import tpu as pltpu
from jax.experimental.pallas import tpu_sc as plsc

N_NODES = 10000
N_EDGES = 320000
IN_F = 128
HID = 256

NPAD = 10240
NC, NS, L = 2, 16, 16
NW = NC * NS
CHUNK = 128
NCHUNK = 80
EPT = CHUNK * NCHUNK
EPAD = EPT * NW
FC = 128


NBUF = 2

SLOWC = 1
NCH_S = 80
NCH_F = 80
TOTCH = NS * (NCH_S + NCH_F)


def _spmm_body(feat_hbm, idx_hbm, out_hbm, cidx, rows, acc, *sems):
    cid = lax.axis_index("c")
    sid = lax.axis_index("s")
    gsem = sems[:NBUF]
    ssem = sems[NBUF:]
    nch = jnp.where(cid == SLOWC, NCH_S, NCH_F)
    cbase = jnp.where(cid == SLOWC, sid * NCH_S, NS * NCH_S + sid * NCH_F)

    def zfill(i, c):
        def zc(k, c2):
            rows[0, i, pl.ds(k * L, L)] = jnp.zeros((L,), jnp.float32)
            return c2
        lax.fori_loop(0, FC // L, zc, 0)
        return c
    lax.fori_loop(0, CHUNK, zfill, 0)

    rpt = NPAD // NS
    base = sid * rpt

    zd = [pltpu.make_async_copy(
        rows.at[0], acc.at[pl.ds(base + k * CHUNK, CHUNK)], ssem[0])
        for k in range(rpt // CHUNK)]
    for d in zd:
        d.start()
    for d in zd:
        d.wait()
    plsc.subcore_barrier()

    for b in range(NBUF):
        pltpu.sync_copy(idx_hbm.at[cbase + b], cidx.at[b])
        pltpu.async_copy(feat_hbm.at[cidx.at[b, 0]], rows.at[b], gsem[b])

    def mloop(t, c):
        for b in range(NBUF):
            pltpu.make_async_copy(
                feat_hbm.at[cidx.at[b, 0]], rows.at[b], gsem[b]).wait()
            pltpu.make_async_copy(
                rows.at[b], acc.at[cidx.at[b, 1]], ssem[b]).start(add=True)
        for b in range(NBUF):
            j = t * NBUF + b
            pltpu.make_async_copy(
                rows.at[b], acc.at[cidx.at[b, 1]], ssem[b]).wait()

            @pl.when(j + NBUF < nch)
            def _():
                pltpu.sync_copy(idx_hbm.at[cbase + j + NBUF], cidx.at[b])
                pltpu.async_copy(feat_hbm.at[cidx.at[b, 0]], rows.at[b], gsem[b])
        return c
    lax.fori_loop(0, nch // NBUF, mloop, 0)
    plsc.subcore_barrier()

    pltpu.sync_copy(acc.at[pl.ds(base, rpt)], out_hbm.at[cid, pl.ds(base, rpt)])


def _make_spmm():
    return pl.kernel(
        _spmm_body,
        out_type=jax.ShapeDtypeStruct((NC, NPAD, FC), jnp.float32),
        mesh=plsc.VectorSubcoreMesh(core_axis_name="c", subcore_axis_name="s"),
        scratch_types=[
            pltpu.VMEM((NBUF, 2, CHUNK), jnp.int32),
            pltpu.VMEM((NBUF, CHUNK, FC), jnp.float32),
            pltpu.VMEM_SHARED((NPAD, FC), jnp.float32),
        ] + [pltpu.SemaphoreType.DMA] * (2 * NBUF),
    )


RB = 1024
NB = NPAD // RB
BE = 4000
NEB = N_EDGES // BE


def _deg_body(s_ref, d_ref, os_ref, od_ref, as_sc, ad_sc):
    i = pl.program_id(0)

    @pl.when(i == 0)
    def _():
        as_sc[...] = jnp.zeros((128, 128), jnp.float32)
        ad_sc[...] = jnp.zeros((128, 128), jnp.float32)

    lane = lax.broadcasted_iota(jnp.int32, (1, 128), 1)

    def onehot_acc(ids, acc):
        hi = lax.shift_right_logical(ids, 7)
        lo = lax.bitwise_and(ids, 127)
        oh_hi = (hi == lane).astype(jnp.float32)
        oh_lo = (lo == lane).astype(jnp.float32)
        acc[...] += lax.dot_general(
            oh_hi, oh_lo, (((0,), (0,)), ((), ())),
            preferred_element_type=jnp.float32)

    onehot_acc(s_ref[...], as_sc)
    onehot_acc(d_ref[...], ad_sc)

    @pl.when(i == NEB - 1)
    def _():
        os_ref[...] = as_sc[...]
        od_ref[...] = ad_sc[...]


def _make_deg():
    return pl.pallas_call(
        _deg_body,
        grid=(NEB,),
        in_specs=[
            pl.BlockSpec((BE, 1), lambda i: (i, 0)),
            pl.BlockSpec((BE, 1), lambda i: (i, 0)),
        ],
        out_specs=[
            pl.BlockSpec((128, 128), lambda i: (0, 0)),
            pl.BlockSpec((128, 128), lambda i: (0, 0)),
        ],
        out_shape=[
            jax.ShapeDtypeStruct((128, 128), jnp.float32),
            jax.ShapeDtypeStruct((128, 128), jnp.float32),
        ],
        scratch_shapes=[
            pltpu.VMEM((128, 128), jnp.float32),
            pltpu.VMEM((128, 128), jnp.float32),
        ],
    )


def _norm(d_ref):
    return lax.rsqrt(jnp.maximum(d_ref[...], 1.0))


def _prep_body(x_ref, dsrc_ref, xn_ref):
    xn_ref[...] = x_ref[...] * _norm(dsrc_ref)


def _layer1_body(agg_ref, ddst_ref, dsrc_ref, w_ref, b_ref, ha_ref, hb_ref):
    agg = (agg_ref[0] + agg_ref[1]) * _norm(ddst_ref)
    h = jnp.maximum(jnp.dot(agg, w_ref[...],
                            preferred_element_type=jnp.float32) + b_ref[...], 0.0)
    hn = h * _norm(dsrc_ref)
    ha_ref[...] = hn[:, :FC]
    hb_ref[...] = hn[:, FC:]


def _layer2_pool_body(agga_ref, aggb_ref, ddst_ref, w_ref, b_ref, wg_ref,
                      bg_ref, wo_ref, bo_ref, out_ref, m_sc, s_sc, r_sc):
    i = pl.program_id(0)
    agg = jnp.concatenate([agga_ref[0] + agga_ref[1],
                           aggb_ref[0] + aggb_ref[1]], axis=1) * _norm(ddst_ref)
    h = jnp.maximum(jnp.dot(agg, w_ref[...],
                            preferred_element_type=jnp.float32) + b_ref[...], 0.0)
    g = jnp.sum(h * wg_ref[...], axis=1) + bg_ref[0, 0]
    rid = i * RB + lax.broadcasted_iota(jnp.int32, (RB,), 0)
    g = jnp.where(rid < N_NODES, g, -1e30)

    @pl.when(i == 0)
    def _():
        m_sc[0, 0] = -1e30
        s_sc[0, 0] = 0.0
        r_sc[...] = jnp.zeros((1, HID), jnp.float32)

    m_old = m_sc[0, 0]
    m_new = jnp.maximum(m_old, jnp.max(g))
    alpha = jnp.exp(m_old - m_new)
    e = jnp.exp(g - m_new)
    s_sc[0, 0] = s_sc[0, 0] * alpha + jnp.sum(e)
    r_sc[...] = r_sc[...] * alpha + jnp.dot(
        e[None, :], h, preferred_element_type=jnp.float32)
    m_sc[0, 0] = m_new

    @pl.when(i == NB - 1)
    def _():
        val = jnp.sum(r_sc[...] * wo_ref[...]) / s_sc[0, 0] + bo_ref[0, 0]
        out_ref[...] = jnp.broadcast_to(val, (1, 1))


def _full(shape):
    return pl.BlockSpec(shape, lambda i: tuple(0 for _ in shape))


def _make_prep():
    return pl.pallas_call(
        _prep_body,
        grid=(NB,),
        in_specs=[
            pl.BlockSpec((RB, IN_F), lambda i: (i, 0)),
            pl.BlockSpec((RB, 1), lambda i: (i, 0)),
        ],
        out_specs=pl.BlockSpec((RB, IN_F), lambda i: (i, 0)),
        out_shape=jax.ShapeDtypeStruct((NPAD, IN_F), jnp.float32),
    )


def _make_layer1():
    return pl.pallas_call(
        _layer1_body,
        grid=(NB,),
        in_specs=[
            pl.BlockSpec((NC, RB, FC), lambda i: (0, i, 0)),
            pl.BlockSpec((RB, 1), lambda i: (i, 0)),
            pl.BlockSpec((RB, 1), lambda i: (i, 0)),
            _full((IN_F, HID)),
            _full((1, HID)),
        ],
        out_specs=[
            pl.BlockSpec((RB, FC), lambda i: (i, 0)),
            pl.BlockSpec((RB, FC), lambda i: (i, 0)),
        ],
        out_shape=[
            jax.ShapeDtypeStruct((NPAD, FC), jnp.float32),
            jax.ShapeDtypeStruct((NPAD, FC), jnp.float32),
        ],
    )


def _make_layer2_pool():
    return pl.pallas_call(
        _layer2_pool_body,
        grid=(NB,),
        in_specs=[
            pl.BlockSpec((NC, RB, FC), lambda i: (0, i, 0)),
            pl.BlockSpec((NC, RB, FC), lambda i: (0, i, 0)),
            pl.BlockSpec((RB, 1), lambda i: (i, 0)),
            _full((HID, HID)),
            _full((1, HID)),
            _full((1, HID)),
            _full((1, 1)),
            _full((1, HID)),
            _full((1, 1)),
        ],
        out_specs=pl.BlockSpec((1, 1), lambda i: (0, 0)),
        out_shape=jax.ShapeDtypeStruct((1, 1), jnp.float32),
        scratch_shapes=[
            pltpu.SMEM((1, 1), jnp.float32),
            pltpu.SMEM((1, 1), jnp.float32),
            pltpu.VMEM((1, HID), jnp.float32),
        ],
    )


@jax.jit
def _run(x, src, dst, W1, b1, W2, b2, Wg, bg, Wo, bo):
    pad_e = EPAD - N_EDGES
    fill = jnp.full((pad_e,), N_NODES, jnp.int32)
    src_p = jnp.concatenate([src, fill]).reshape(TOTCH, 1, CHUNK)
    dst_p = jnp.concatenate([dst, fill]).reshape(TOTCH, 1, CHUNK)
    idx_p = jnp.concatenate([src_p, dst_p], axis=1)
    x_p = jnp.pad(x, ((0, NPAD - N_NODES), (0, 0)))

    ds_hl, dd_hl = _make_deg()(src.reshape(N_EDGES, 1), dst.reshape(N_EDGES, 1))
    dsrc = ds_hl.reshape(128 * 128)[:NPAD, None]
    ddst = dd_hl.reshape(128 * 128)[:NPAD, None]

    spmm = _make_spmm()
    xn = _make_prep()(x_p, dsrc)
    agg1 = spmm(xn, idx_p)
    h1a, h1b = _make_layer1()(agg1, ddst, dsrc, W1, b1.reshape(1, HID))
    agg2a = spmm(h1a, idx_p)
    h1b_seq, agg2a = lax.optimization_barrier((h1b, agg2a))
    agg2b = spmm(h1b_seq, idx_p)
    out = _make_layer2_pool()(
        agg2a, agg2b, ddst, W2, b2.reshape(1, HID),
        Wg.reshape(1, HID), bg.reshape(1, 1),
        Wo.reshape(1, HID), bo.reshape(1, 1))
    return out


def kernel(x, edge_index, W1, b1, W2, b2, Wg, bg, Wo, bo):
    ei = edge_index.astype(jnp.int32)
    return _run(x, ei[0], ei[1], W1, b1, W2, b2, Wg, bg, Wo, bo)
